# baseline (speedup 1.0000x reference)
import jax
import jax.numpy as jnp
from jax import lax
from jax.experimental import pallas as pl
from jax.experimental.pallas import tpu as pltpu
from jax.experimental.pallas import tpu_sc as plsc

_NUM_EXPERTS = 64
_TOP_K = 2
_D = 768
_B = 4
_S = 8192

_NC = 2
_NS = 16
_NW = _NC * _NS
_ROWS_W = _B * _S // _NW
_CHUNK = 64
_NCHUNK = _ROWS_W // _CHUNK
_NPANEL = _D // 128
_UNROLL = 4


def _accumulate(buf, accv):
    for g in range(_NPANEL):
        base = g * 128
        acc = tuple(accv[pl.ds(base + j * 16, 16)] for j in range(8))

        def rbody(i, a, buf=buf, base=base):
            a = list(a)
            for u in range(_UNROLL):
                r = i * _UNROLL + u
                for j in range(8):
                    a[j] = a[j] + buf[r, pl.ds(base + j * 16, 16)]
            return tuple(a)

        acc = lax.fori_loop(0, _CHUNK // _UNROLL, rbody, acc)
        for j in range(8):
            accv[pl.ds(base + j * 16, 16)] = acc[j]


def _rowsum_body(x_hbm, out_hbm, buf0, buf1, accv, sem0, sem1):
    wid = lax.axis_index("s") * _NC + lax.axis_index("c")
    row0 = wid * _ROWS_W

    def start(k, buf, sem):
        return pltpu.async_copy(
            x_hbm.at[pl.ds(row0 + k * _CHUNK, _CHUNK)], buf, sem)

    zero = jnp.zeros((16,), jnp.float32)
    for j in range(_D // 16):
        accv[pl.ds(j * 16, 16)] = zero

    start(0, buf0, sem0)

    def pair(kp, carry):
        k0 = 2 * kp
        start(k0 + 1, buf1, sem1)
        pltpu.make_async_copy(
            x_hbm.at[pl.ds(row0 + k0 * _CHUNK, _CHUNK)], buf0, sem0).wait()
        _accumulate(buf0, accv)

        @pl.when(k0 + 2 < _NCHUNK)
        def _():
            start(k0 + 2, buf0, sem0)

        pltpu.make_async_copy(
            x_hbm.at[pl.ds(row0 + (k0 + 1) * _CHUNK, _CHUNK)],
            buf1, sem1).wait()
        _accumulate(buf1, accv)
        return carry

    lax.fori_loop(0, _NCHUNK // 2, pair, 0)
    pltpu.sync_copy(accv, out_hbm.at[wid])


_rowsum_cache = []


def _get_rowsum():
    if not _rowsum_cache:
        _rowsum_cache.append(pl.kernel(
            _rowsum_body,
            out_type=jax.ShapeDtypeStruct((_NW, _D), jnp.float32),
            mesh=plsc.VectorSubcoreMesh(core_axis_name="c",
                                        subcore_axis_name="s"),
            scratch_types=[
                pltpu.VMEM((_CHUNK, _D), jnp.float32),
                pltpu.VMEM((_CHUNK, _D), jnp.float32),
                pltpu.VMEM((_D,), jnp.float32),
                pltpu.SemaphoreType.DMA,
                pltpu.SemaphoreType.DMA,
            ],
        ))
    return _rowsum_cache[0]


def _gate_body(part_ref, w_ref, gain_ref, comb_ref, idx_ref, top_ref):
    parts = part_ref[...]
    r4 = lax.broadcasted_iota(jnp.int32, (_B, _NW), 0)
    c32 = lax.broadcasted_iota(jnp.int32, (_B, _NW), 1)
    sel = jnp.where(c32 // (_NW // _B) == r4, 1.0, 0.0)
    sums = lax.dot_general(sel, parts, (((1,), (0,)), ((), ())),
                           preferred_element_type=jnp.float32)
    s = sums * (1.0 / _S)
    w = w_ref[...]
    g = gain_ref[...]
    weff = (w - jnp.mean(w, axis=1, keepdims=True)) * g
    scores = lax.dot_general(
        s, weff, (((1,), (1,)), ((), ())),
        preferred_element_type=jnp.float32)
    m = jnp.max(scores, axis=1, keepdims=True)
    e = jnp.exp(scores - m)
    p = e / jnp.sum(e, axis=1, keepdims=True)
    eidx = lax.broadcasted_iota(jnp.int32, p.shape, 1)
    big = jnp.int32(2 ** 30)
    p1 = jnp.max(p, axis=1, keepdims=True)
    i1 = jnp.min(jnp.where(p == p1, eidx, big), axis=1, keepdims=True)
    pm = jnp.where(eidx == i1, -jnp.inf, p)
    p2 = jnp.max(pm, axis=1, keepdims=True)
    i2 = jnp.min(jnp.where(pm == p2, eidx, big), axis=1, keepdims=True)
    den = p1 + p2 + 1e-9
    comb_ref[...] = (jnp.where(eidx == i1, p1 / den, 0.0)
                     + jnp.where(eidx == i2, p2 / den, 0.0))
    kidx = lax.broadcasted_iota(jnp.int32, (_B, _TOP_K), 1)
    idx_ref[...] = jnp.where(kidx == 0,
                             jnp.broadcast_to(i1, (_B, _TOP_K)),
                             jnp.broadcast_to(i2, (_B, _TOP_K)))
    top_ref[...] = jnp.where(kidx == 0,
                             jnp.broadcast_to(p1, (_B, _TOP_K)),
                             jnp.broadcast_to(p2, (_B, _TOP_K)))


_gate = pl.pallas_call(
    _gate_body,
    out_shape=(
        jax.ShapeDtypeStruct((_B, _NUM_EXPERTS), jnp.float32),
        jax.ShapeDtypeStruct((_B, _TOP_K), jnp.int32),
        jax.ShapeDtypeStruct((_B, _TOP_K), jnp.float32),
    ),
)


def kernel(x, W, gain):
    parts = _get_rowsum()(x.reshape(_B * _S, _D))
    return _gate(parts, W, gain.reshape(_NUM_EXPERTS, 1))
